# baseline (speedup 1.0000x reference)
import jax
import jax.numpy as jnp
from jax.experimental import pallas as pl
from jax.experimental.pallas import tpu as pltpu

N = 10000
H = 64
F_IN = 128
BM = 200
NB = N // BM
OUT_AGG = 10


def _body(x_ref, sadj_ref, fadj_ref, w1_ref, wc_ref, w2_ref,
          b1_ref, bc_ref, b2_ref, a1_ref, ab1_ref, a2_ref,
          out_ref, e1_ref, c1_ref, c2_ref, e2_ref,
          sa_scr, sb_scr):
    i = pl.program_id(0)

    @pl.when(i == 0)
    def _supports():
        xb = x_ref[...]
        w1 = w1_ref[...]
        wc = wc_ref[...]
        w2 = w2_ref[...]
        s1 = jnp.dot(xb, w1 * w1, preferred_element_type=jnp.float32)
        sc = jnp.dot(xb, wc * wc, preferred_element_type=jnp.float32)
        s2 = jnp.dot(xb, w2 * w2, preferred_element_type=jnp.float32)
        sa_scr[...] = jnp.concatenate([s1, sc], axis=1).astype(jnp.bfloat16)
        sb_scr[...] = jnp.concatenate([sc, s2], axis=1).astype(jnp.bfloat16)

    acc_s = jnp.dot(sadj_ref[...].astype(jnp.bfloat16), sa_scr[...],
                    preferred_element_type=jnp.float32)
    acc_f = jnp.dot(fadj_ref[...].astype(jnp.bfloat16), sb_scr[...],
                    preferred_element_type=jnp.float32)

    emb1 = acc_s[:, :H] + b1_ref[...]
    com1 = acc_s[:, H:] + bc_ref[...]
    com2 = acc_f[:, :H] + bc_ref[...]
    emb2 = acc_f[:, H:] + b2_ref[...]
    xcom = com1 * com2

    a1 = a1_ref[...]
    ab1 = ab1_ref[...]
    a2 = a2_ref[...]

    def logit(e):
        h = jnp.tanh(jnp.dot(e, a1, preferred_element_type=jnp.float32) + ab1)
        return jnp.sum(h * a2, axis=1, keepdims=True)

    w1l = logit(emb1)
    w2l = logit(emb2)
    w3l = logit(xcom)
    m = jnp.maximum(jnp.maximum(w1l, w2l), w3l)
    p1 = jnp.exp(w1l - m)
    p2 = jnp.exp(w2l - m)
    p3 = jnp.exp(w3l - m)
    denom = p1 + p2 + p3
    r = pl.ds((i % OUT_AGG) * BM, BM)
    out_ref[r, :] = (p1 * emb1 + p2 * emb2 + p3 * xcom) / denom
    e1_ref[r, :] = emb1
    c1_ref[r, :] = com1
    c2_ref[r, :] = com2
    e2_ref[r, :] = emb2


def kernel(x, sadj, fadj, W1, b1, W2, b2, Wc, bc, A1, ab1, A2):
    f32 = jnp.float32
    bf16 = jnp.bfloat16

    b1r = b1.reshape(1, H)
    bcr = bc.reshape(1, H)
    b2r = b2.reshape(1, H)
    ab1r = ab1.reshape(1, -1)
    a2r = A2.reshape(1, -1)

    full = lambda i: (0, 0)
    out_spec = pl.BlockSpec((BM * OUT_AGG, H), lambda i: (i // OUT_AGG, 0))
    out_shape = jax.ShapeDtypeStruct((N, H), f32)

    outs = pl.pallas_call(
        _body,
        grid=(NB,),
        in_specs=[
            pl.BlockSpec((N, F_IN), full),
            pl.BlockSpec((BM, N), lambda i: (i, 0)),
            pl.BlockSpec((BM, N), lambda i: (i, 0)),
            pl.BlockSpec((F_IN, H), full),
            pl.BlockSpec((F_IN, H), full),
            pl.BlockSpec((F_IN, H), full),
            pl.BlockSpec((1, H), full),
            pl.BlockSpec((1, H), full),
            pl.BlockSpec((1, H), full),
            pl.BlockSpec(A1.shape, full),
            pl.BlockSpec((1, ab1.shape[0]), full),
            pl.BlockSpec((1, A2.shape[0]), full),
        ],
        out_specs=[out_spec] * 5,
        out_shape=[out_shape] * 5,
        scratch_shapes=[
            pltpu.VMEM((N, 2 * H), bf16),
            pltpu.VMEM((N, 2 * H), bf16),
        ],
        compiler_params=pltpu.CompilerParams(
            dimension_semantics=("arbitrary",),
        ),
    )(x, sadj, fadj, W1, Wc, W2, b1r, bcr, b2r, A1, ab1r, a2r)

    output, emb1, com1, com2, emb2 = outs
    return (output, emb1, com1, com2, emb2)


# speedup vs baseline: 1.0170x; 1.0170x over previous
import jax
import jax.numpy as jnp
from jax.experimental import pallas as pl
from jax.experimental.pallas import tpu as pltpu

N = 10000
H = 64
F_IN = 128
BM = 200
NB = N // BM
OUT_AGG = 5


def _body(x_ref, sadj_ref, fadj_ref, w1_ref, wc_ref, w2_ref,
          b1_ref, bc_ref, b2_ref, a1_ref, ab1_ref, a2_ref,
          out_ref, e1_ref, c1_ref, c2_ref, e2_ref,
          sa_scr, sb_scr):
    i = pl.program_id(0)

    @pl.when(i == 0)
    def _supports():
        xb = x_ref[...]
        w1 = w1_ref[...]
        wc = wc_ref[...]
        w2 = w2_ref[...]
        s1 = jnp.dot(xb, w1 * w1, preferred_element_type=jnp.float32)
        sc = jnp.dot(xb, wc * wc, preferred_element_type=jnp.float32)
        s2 = jnp.dot(xb, w2 * w2, preferred_element_type=jnp.float32)
        sa_scr[...] = jnp.concatenate([s1, sc], axis=1).astype(jnp.bfloat16)
        sb_scr[...] = jnp.concatenate([sc, s2], axis=1).astype(jnp.bfloat16)

    @pl.when(i > 0)
    def _rows():
        acc_s = jnp.dot(sadj_ref[...].astype(jnp.bfloat16), sa_scr[...],
                        preferred_element_type=jnp.float32)
        acc_f = jnp.dot(fadj_ref[...].astype(jnp.bfloat16), sb_scr[...],
                        preferred_element_type=jnp.float32)

        emb1 = acc_s[:, :H] + b1_ref[...]
        com1 = acc_s[:, H:] + bc_ref[...]
        com2 = acc_f[:, :H] + bc_ref[...]
        emb2 = acc_f[:, H:] + b2_ref[...]
        xcom = com1 * com2

        a1 = a1_ref[...]
        ab1 = ab1_ref[...]
        a2 = a2_ref[...]

        def logit(e):
            h = jnp.tanh(jnp.dot(e, a1, preferred_element_type=jnp.float32)
                         + ab1)
            return jnp.sum(h * a2, axis=1, keepdims=True)

        w1l = logit(emb1)
        w2l = logit(emb2)
        w3l = logit(xcom)
        m = jnp.maximum(jnp.maximum(w1l, w2l), w3l)
        p1 = jnp.exp(w1l - m)
        p2 = jnp.exp(w2l - m)
        p3 = jnp.exp(w3l - m)
        denom = p1 + p2 + p3
        r = pl.ds(((i - 1) % OUT_AGG) * BM, BM)
        out_ref[r, :] = (p1 * emb1 + p2 * emb2 + p3 * xcom) / denom
        e1_ref[r, :] = emb1
        c1_ref[r, :] = com1
        c2_ref[r, :] = com2
        e2_ref[r, :] = emb2


def kernel(x, sadj, fadj, W1, b1, W2, b2, Wc, bc, A1, ab1, A2):
    f32 = jnp.float32
    bf16 = jnp.bfloat16

    b1r = b1.reshape(1, H)
    bcr = bc.reshape(1, H)
    b2r = b2.reshape(1, H)
    ab1r = ab1.reshape(1, -1)
    a2r = A2.reshape(1, -1)

    full = lambda i: (0, 0)
    out_spec = pl.BlockSpec(
        (BM * OUT_AGG, H), lambda i: (jnp.maximum(i - 1, 0) // OUT_AGG, 0))
    out_shape = jax.ShapeDtypeStruct((N, H), f32)

    row = lambda i: (jnp.maximum(i - 1, 0), 0)
    outs = pl.pallas_call(
        _body,
        grid=(NB + 1,),
        in_specs=[
            pl.BlockSpec((N, F_IN), full),
            pl.BlockSpec((BM, N), row),
            pl.BlockSpec((BM, N), row),
            pl.BlockSpec((F_IN, H), full),
            pl.BlockSpec((F_IN, H), full),
            pl.BlockSpec((F_IN, H), full),
            pl.BlockSpec((1, H), full),
            pl.BlockSpec((1, H), full),
            pl.BlockSpec((1, H), full),
            pl.BlockSpec(A1.shape, full),
            pl.BlockSpec((1, ab1.shape[0]), full),
            pl.BlockSpec((1, A2.shape[0]), full),
        ],
        out_specs=[out_spec] * 5,
        out_shape=[out_shape] * 5,
        scratch_shapes=[
            pltpu.VMEM((N, 2 * H), bf16),
            pltpu.VMEM((N, 2 * H), bf16),
        ],
        compiler_params=pltpu.CompilerParams(
            dimension_semantics=("arbitrary",),
        ),
    )(x, sadj, fadj, W1, Wc, W2, b1r, bcr, b2r, A1, ab1r, a2r)

    output, emb1, com1, com2, emb2 = outs
    return (output, emb1, com1, com2, emb2)
